# baseline (speedup 1.0000x reference)
import functools

import jax
import jax.numpy as jnp
from jax.experimental import pallas as pl


def _matmul_block_kernel(x_ref, e_ref, o_ref):
    o_ref[...] = jnp.dot(x_ref[...], e_ref[...],
                         preferred_element_type=jnp.float32)


@functools.partial(jax.jit, static_argnames=("bm",))
def _stream_matmul(x, e, bm):
    m, d = x.shape
    _, k = e.shape
    grid = (m // bm,)
    return pl.pallas_call(
        _matmul_block_kernel,
        grid=grid,
        in_specs=[
            pl.BlockSpec((bm, d), lambda i: (i, 0)),
            pl.BlockSpec((d, k), lambda i: (0, 0)),
        ],
        out_specs=pl.BlockSpec((bm, k), lambda i: (i, 0)),
        out_shape=jax.ShapeDtypeStruct((m, k), jnp.float32),
    )(x, e)


def kernel(U, V, Eu, Ev):
    P = _stream_matmul(U, Eu, 512)
    Q = _stream_matmul(V, Ev, 512)
    return (P, Q)


# speedup vs baseline: 1.1279x; 1.1279x over previous
import jax
import jax.numpy as jnp
from jax.experimental import pallas as pl


def _matmul_block_kernel(x_ref, e_ref, o_ref):
    o_ref[...] = jnp.dot(x_ref[...], e_ref[...],
                         preferred_element_type=jnp.float32)


def _stream_matmul(x, e, bm):
    m, d = x.shape
    _, k = e.shape
    grid = (m // bm,)
    return pl.pallas_call(
        _matmul_block_kernel,
        grid=grid,
        in_specs=[
            pl.BlockSpec((bm, d), lambda i: (i, 0)),
            pl.BlockSpec((d, k), lambda i: (0, 0)),
        ],
        out_specs=pl.BlockSpec((bm, k), lambda i: (i, 0)),
        out_shape=jax.ShapeDtypeStruct((m, k), jnp.float32),
    )(x, e)


def kernel(U, V, Eu, Ev):
    P = _stream_matmul(U, Eu, 2048)
    Q = _stream_matmul(V, Ev, 2048)
    return (P, Q)


# speedup vs baseline: 4.3407x; 3.8484x over previous
import jax
import jax.numpy as jnp
from jax.experimental import pallas as pl


def _matmul_block_kernel(e_ref, x_ref, o_ref):
    o_ref[...] = jnp.dot(e_ref[...], x_ref[...],
                         preferred_element_type=jnp.float32)


def _stream_matmul_t(et, xt, bn):
    k, d = et.shape
    _, n = xt.shape
    grid = (n // bn,)
    return pl.pallas_call(
        _matmul_block_kernel,
        grid=grid,
        in_specs=[
            pl.BlockSpec((k, d), lambda i: (0, 0)),
            pl.BlockSpec((d, bn), lambda i: (0, i)),
        ],
        out_specs=pl.BlockSpec((k, bn), lambda i: (0, i)),
        out_shape=jax.ShapeDtypeStruct((k, n), jnp.float32),
    )(et, xt)


def kernel(U, V, Eu, Ev):
    Pt = _stream_matmul_t(Eu.T, U.T, 2048)
    Qt = _stream_matmul_t(Ev.T, V.T, 2048)
    return (Pt.T, Qt.T)
